# baseline (speedup 1.0000x reference)
import functools

import jax
import jax.numpy as jnp
from jax import lax
from jax.experimental import pallas as pl
from jax.experimental.pallas import tpu as pltpu
from jax.experimental.pallas import tpu_sc as plsc


def _proj_body(x_ref, w_ref, b_ref, o_ref):
    res = jnp.dot(x_ref[...], w_ref[...], preferred_element_type=jnp.float32)
    res = jnp.maximum(res + b_ref[...], 0.0)
    n = res.shape[0]
    hh = res.shape[1] // 2
    o_ref[pl.ds(0, n)] = res[:, :hh]
    o_ref[pl.ds(n, n)] = res[:, hh:]


def _mlp_body(ha_ref, hb_ref, fa_ref, fb_ref, eps_ref, w1_ref, b1_ref,
              w2_ref, b2_ref, o_ref):
    hblk = jnp.concatenate([ha_ref[...], hb_ref[...]], axis=1)
    agg = jnp.concatenate([fa_ref[...], fb_ref[...]], axis=1)
    t = (1.0 + eps_ref[0, 0]) * hblk + agg
    u = jnp.dot(t, w1_ref[...], preferred_element_type=jnp.float32) + b1_ref[...]
    u = jnp.maximum(u, 0.0)
    o_ref[...] = jnp.dot(u, w2_ref[...], preferred_element_type=jnp.float32) + b2_ref[...]


def _bn_body(v_ref, g_ref, b_ref, o_ref):
    v = v_ref[...]
    n = v.shape[0]
    hh = v.shape[1] // 2
    mean = jnp.sum(v, axis=0, keepdims=True) / n
    c = v - mean
    var = jnp.sum(c * c, axis=0, keepdims=True) / n
    out = g_ref[...] * c * lax.rsqrt(var + 1e-5) + b_ref[...]
    out = jnp.maximum(out, 0.0)
    o_ref[pl.ds(0, n)] = out[:, :hh]
    o_ref[pl.ds(n, n)] = out[:, hh:]


def _pool_body(seed_ref, hcat_ref, w1_ref, w2_ref, wa_ref, cw_ref, cb_ref,
               alpha_ref, logit_ref):
    hc = hcat_ref[...]
    n = hc.shape[0] // 2
    h = jnp.concatenate([hc[:n], hc[n:]], axis=1)
    hmean = jnp.sum(h, axis=0, keepdims=True) / n
    seed = seed_ref[0]
    in_range = (seed >= 0) & (seed < n)
    safe = jnp.clip(seed, 0, n - 1)
    hrow = jnp.concatenate(
        [hcat_ref[pl.ds(safe, 1), :], hcat_ref[pl.ds(safe + n, 1), :]],
        axis=1)
    hs = jnp.where(in_range, hrow, hmean)
    z = jnp.tanh(
        jnp.dot(h, w1_ref[...], preferred_element_type=jnp.float32)
        + jnp.dot(hs, w2_ref[...], preferred_element_type=jnp.float32))
    e = jnp.dot(z, wa_ref[...], preferred_element_type=jnp.float32)
    e = jnp.where(e >= 0.0, e, 0.2 * e)
    m = jnp.max(e)
    a = jnp.exp(e - m)
    alpha = a / jnp.sum(a)
    alpha_ref[...] = alpha
    r = jnp.sum(alpha * h, axis=0, keepdims=True)
    logit_ref[...] = (
        jnp.dot(r, cw_ref[...], preferred_element_type=jnp.float32) + cb_ref[...])


def _proj(x, w, b):
    n, _ = x.shape
    h = w.shape[1]
    return pl.pallas_call(
        _proj_body,
        out_shape=jax.ShapeDtypeStruct((2 * n, h // 2), jnp.float32),
    )(x, w, b.reshape(1, h))


def _mlp(hcat, flat, eps_i, w1, b1, w2, b2):
    n2, hh = hcat.shape
    n = n2 // 2
    h = 2 * hh
    br = 400
    nb = n // br
    grid = (nb,)
    return pl.pallas_call(
        _mlp_body,
        grid=grid,
        in_specs=[
            pl.BlockSpec((br, hh), lambda i: (i, 0)),
            pl.BlockSpec((br, hh), lambda i: (nb + i, 0)),
            pl.BlockSpec((br, hh), lambda i: (i, 0)),
            pl.BlockSpec((br, hh), lambda i: (nb + i, 0)),
            pl.BlockSpec((1, 1), lambda i: (0, 0), memory_space=pltpu.SMEM),
            pl.BlockSpec((h, h), lambda i: (0, 0)),
            pl.BlockSpec((1, h), lambda i: (0, 0)),
            pl.BlockSpec((h, h), lambda i: (0, 0)),
            pl.BlockSpec((1, h), lambda i: (0, 0)),
        ],
        out_specs=pl.BlockSpec((br, h), lambda i: (i, 0)),
        out_shape=jax.ShapeDtypeStruct((n, h), jnp.float32),
    )(hcat, hcat, flat, flat, eps_i.reshape(1, 1), w1, b1.reshape(1, h),
      w2, b2.reshape(1, h))


def _bn(v, gamma, beta):
    n, h = v.shape
    return pl.pallas_call(
        _bn_body,
        out_shape=jax.ShapeDtypeStruct((2 * n, h // 2), jnp.float32),
    )(v, gamma.reshape(1, h), beta.reshape(1, h))


def _pool(seed, hcat, w1, w2, wa, cw, cb):
    n = hcat.shape[0] // 2
    return pl.pallas_call(
        _pool_body,
        in_specs=[
            pl.BlockSpec(memory_space=pltpu.SMEM),
            pl.BlockSpec(),
            pl.BlockSpec(),
            pl.BlockSpec(),
            pl.BlockSpec(),
            pl.BlockSpec(),
            pl.BlockSpec(),
        ],
        out_shape=(
            jax.ShapeDtypeStruct((n, 1), jnp.float32),
            jax.ShapeDtypeStruct((1, 1), jnp.float32),
        ),
    )(seed, hcat, w1, w2, wa, cw, cb.reshape(1, 1))


def _segment_sum_sc(hcat, packed, n):
    n2, hh = hcat.shape
    nck, ch = packed.shape
    e = nck * ch // 2
    info = plsc.get_sparse_core_info()
    nc, ns = info.num_cores, info.num_subcores
    cpt = e // ns // ch
    nbuf = 5
    rpt = -(-n // ns)
    rpt = -(-rpt // ch) * ch
    np_ = rpt * ns
    nz = rpt // ch

    t1 = n - (ns - 1) * rpt
    t2 = rpt - t1

    mesh = plsc.VectorSubcoreMesh(core_axis_name="c", subcore_axis_name="s")

    @functools.partial(
        pl.kernel,
        out_type=jax.ShapeDtypeStruct((nc * np_, hh), jnp.float32),
        mesh=mesh,
        scratch_types=(
            [pltpu.VMEM((cpt, ch), jnp.int32)]
            + [pltpu.VMEM((ch,), jnp.int32) for _ in range(2 * nbuf)]
            + [pltpu.VMEM((ch, hh), jnp.float32) for _ in range(nbuf)]
            + [pltpu.VMEM_SHARED((np_, hh), jnp.float32)]
            + [pltpu.SemaphoreType.DMA for _ in range(nbuf)]
        ),
        compiler_params=pltpu.CompilerParams(use_tc_tiling_on_sc=False),
    )
    def seg(h_hbm, packed_hbm, out_hbm, packed_v, *rest):
        srcb = rest[:nbuf]
        dstb = rest[nbuf:2 * nbuf]
        bufs = rest[2 * nbuf:3 * nbuf]
        acc = rest[3 * nbuf]
        sems = rest[3 * nbuf + 1:]
        cid = lax.axis_index("c")
        sid = lax.axis_index("s")

        pltpu.sync_copy(packed_hbm.at[pl.ds((cid * ns + sid) * cpt, cpt)],
                        packed_v)

        zero16 = jnp.zeros((16,), jnp.float32)

        def zrow(r, carry):
            for c in range(hh // 16):
                bufs[0][r, pl.ds(c * 16, 16)] = zero16
            return carry

        lax.fori_loop(0, ch, zrow, 0)
        for j in range(nz):
            pltpu.sync_copy(bufs[0], acc.at[pl.ds(sid * rpt + j * ch, ch)])
        plsc.subcore_barrier()

        def eloop(t, carry):
            g = t * nbuf
            descs = []
            for k in range(nbuf):
                for j in range(ch // 16):
                    p = packed_v[g + k, pl.ds(j * 16, 16)]
                    srcb[k][pl.ds(j * 16, 16)] = p & 0xFFFF
                    dstb[k][pl.ds(j * 16, 16)] = lax.shift_right_logical(p, 16)
                descs.append(
                    pltpu.async_copy(h_hbm.at[srcb[k]], bufs[k], sems[k]))
            for k in range(nbuf):
                descs[k].wait()
                pltpu.sync_copy(bufs[k], acc.at[dstb[k]], add=True)
            return carry

        lax.fori_loop(0, cpt // nbuf, eloop, 0)
        plsc.subcore_barrier()

        row0 = sid * rpt
        pltpu.sync_copy(acc.at[pl.ds(row0, rpt)],
                        out_hbm.at[pl.ds(cid * np_ + row0, rpt)])

    flat2 = seg(hcat, packed)
    return jnp.concatenate([flat2[:n], flat2[np_:np_ + n]], axis=0)


def kernel(x, edge_index, seed_idx, proj_W, proj_b, eps, lin1_W, lin1_b,
           lin2_W, lin2_b, bn_gamma, bn_beta, W1, W2, w_attn, cls_W, cls_b):
    n = x.shape[0]
    num_layers = eps.shape[0]
    ei = edge_index.astype(jnp.int32)
    src = ei[0]
    dst = ei[1]
    ch = 80
    e = src.shape[0]
    packed = (jnp.concatenate([src, src + n])
              | (jnp.concatenate([dst, dst]) << 16)).reshape(2 * e // ch, ch)
    seed = jnp.asarray(seed_idx, jnp.int32).reshape(1)

    hcat = _proj(x, proj_W, proj_b)
    for i in range(num_layers):
        flat = _segment_sum_sc(hcat, packed, n)
        v = _mlp(hcat, flat, eps[i], lin1_W[i], lin1_b[i], lin2_W[i],
                 lin2_b[i])
        hcat = _bn(v, bn_gamma[i], bn_beta[i])

    alpha2, logit2 = _pool(seed, hcat, W1, W2, w_attn, cls_W, cls_b)
    return (logit2.reshape(1), alpha2[:, 0])


# speedup vs baseline: 1.0486x; 1.0486x over previous
import functools

import jax
import jax.numpy as jnp
from jax import lax
from jax.experimental import pallas as pl
from jax.experimental.pallas import tpu as pltpu
from jax.experimental.pallas import tpu_sc as plsc


def _proj_body(x_ref, w_ref, b_ref, o_ref):
    res = jnp.dot(x_ref[...], w_ref[...], preferred_element_type=jnp.float32)
    res = jnp.maximum(res + b_ref[...], 0.0)
    n = res.shape[0]
    hh = res.shape[1] // 2
    o_ref[pl.ds(0, n)] = res[:, :hh]
    o_ref[pl.ds(n, n)] = res[:, hh:]


def _layer_body(hcat_ref, flat_ref, eps_ref, w1_ref, b1_ref, w2_ref, b2_ref,
                g_ref, bt_ref, o_ref):
    n = hcat_ref.shape[0] // 2
    hh = hcat_ref.shape[1]
    hblk = jnp.concatenate(
        [hcat_ref[pl.ds(0, n)], hcat_ref[pl.ds(n, n)]], axis=1)
    agg = jnp.concatenate(
        [flat_ref[pl.ds(0, n)], flat_ref[pl.ds(n, n)]], axis=1)
    t = (1.0 + eps_ref[0]) * hblk + agg
    u = jnp.dot(t, w1_ref[...], preferred_element_type=jnp.float32) + b1_ref[...]
    u = jnp.maximum(u, 0.0)
    v = jnp.dot(u, w2_ref[...], preferred_element_type=jnp.float32) + b2_ref[...]
    mean = jnp.sum(v, axis=0, keepdims=True) / n
    c = v - mean
    var = jnp.sum(c * c, axis=0, keepdims=True) / n
    out = g_ref[...] * c * lax.rsqrt(var + 1e-5) + bt_ref[...]
    out = jnp.maximum(out, 0.0)
    o_ref[pl.ds(0, n)] = out[:, :hh]
    o_ref[pl.ds(n, n)] = out[:, hh:]


def _pool_body(seed_ref, hcat_ref, w1_ref, w2_ref, wa_ref, cw_ref, cb_ref,
               alpha_ref, logit_ref):
    hc = hcat_ref[...]
    n = hc.shape[0] // 2
    h = jnp.concatenate([hc[:n], hc[n:]], axis=1)
    hmean = jnp.sum(h, axis=0, keepdims=True) / n
    seed = seed_ref[0]
    in_range = (seed >= 0) & (seed < n)
    safe = jnp.clip(seed, 0, n - 1)
    hrow = jnp.concatenate(
        [hcat_ref[pl.ds(safe, 1), :], hcat_ref[pl.ds(safe + n, 1), :]],
        axis=1)
    hs = jnp.where(in_range, hrow, hmean)
    z = jnp.tanh(
        jnp.dot(h, w1_ref[...], preferred_element_type=jnp.float32)
        + jnp.dot(hs, w2_ref[...], preferred_element_type=jnp.float32))
    e = jnp.dot(z, wa_ref[...], preferred_element_type=jnp.float32)
    e = jnp.where(e >= 0.0, e, 0.2 * e)
    m = jnp.max(e)
    a = jnp.exp(e - m)
    alpha = a / jnp.sum(a)
    alpha_ref[...] = alpha
    r = jnp.sum(alpha * h, axis=0, keepdims=True)
    logit_ref[...] = (
        jnp.dot(r, cw_ref[...], preferred_element_type=jnp.float32) + cb_ref[...])


def _proj(x, w, b):
    n, _ = x.shape
    h = w.shape[1]
    return pl.pallas_call(
        _proj_body,
        out_shape=jax.ShapeDtypeStruct((2 * n, h // 2), jnp.float32),
    )(x, w, b.reshape(1, h))


def _layer(hcat, flat, eps_i, w1, b1, w2, b2, gamma, beta):
    n2, hh = hcat.shape
    h = 2 * hh
    return pl.pallas_call(
        _layer_body,
        in_specs=[
            pl.BlockSpec(),
            pl.BlockSpec(),
            pl.BlockSpec(memory_space=pltpu.SMEM),
            pl.BlockSpec(),
            pl.BlockSpec(),
            pl.BlockSpec(),
            pl.BlockSpec(),
            pl.BlockSpec(),
            pl.BlockSpec(),
        ],
        out_shape=jax.ShapeDtypeStruct((n2, hh), jnp.float32),
    )(hcat, flat, eps_i.reshape(1), w1, b1.reshape(1, h), w2,
      b2.reshape(1, h), gamma.reshape(1, h), beta.reshape(1, h))


def _pool(seed, hcat, w1, w2, wa, cw, cb):
    n = hcat.shape[0] // 2
    return pl.pallas_call(
        _pool_body,
        in_specs=[
            pl.BlockSpec(memory_space=pltpu.SMEM),
            pl.BlockSpec(),
            pl.BlockSpec(),
            pl.BlockSpec(),
            pl.BlockSpec(),
            pl.BlockSpec(),
            pl.BlockSpec(),
        ],
        out_shape=(
            jax.ShapeDtypeStruct((n, 1), jnp.float32),
            jax.ShapeDtypeStruct((1, 1), jnp.float32),
        ),
    )(seed, hcat, w1, w2, wa, cw, cb.reshape(1, 1))


def _segment_sum_sc(hcat, packed, n):
    n2, hh = hcat.shape
    nck, ch = packed.shape
    e = nck * ch // 2
    info = plsc.get_sparse_core_info()
    nc, ns = info.num_cores, info.num_subcores
    cpt = e // ns // ch
    nbuf = 5
    rpt = -(-n // ns)
    rpt = -(-rpt // ch) * ch
    np_ = rpt * ns
    nz = rpt // ch

    t1 = n - (ns - 1) * rpt
    t2 = rpt - t1

    mesh = plsc.VectorSubcoreMesh(core_axis_name="c", subcore_axis_name="s")

    @functools.partial(
        pl.kernel,
        out_type=jax.ShapeDtypeStruct((nc * np_, hh), jnp.float32),
        mesh=mesh,
        scratch_types=(
            [pltpu.VMEM((cpt, ch), jnp.int32)]
            + [pltpu.VMEM((ch,), jnp.int32) for _ in range(2 * nbuf)]
            + [pltpu.VMEM((ch, hh), jnp.float32) for _ in range(nbuf)]
            + [pltpu.VMEM_SHARED((np_, hh), jnp.float32)]
            + [pltpu.SemaphoreType.DMA for _ in range(nbuf)]
        ),
        compiler_params=pltpu.CompilerParams(use_tc_tiling_on_sc=False),
    )
    def seg(h_hbm, packed_hbm, out_hbm, packed_v, *rest):
        srcb = rest[:nbuf]
        dstb = rest[nbuf:2 * nbuf]
        bufs = rest[2 * nbuf:3 * nbuf]
        acc = rest[3 * nbuf]
        sems = rest[3 * nbuf + 1:]
        cid = lax.axis_index("c")
        sid = lax.axis_index("s")

        pltpu.sync_copy(packed_hbm.at[pl.ds((cid * ns + sid) * cpt, cpt)],
                        packed_v)

        zero16 = jnp.zeros((16,), jnp.float32)

        def zrow(r, carry):
            for c in range(hh // 16):
                bufs[0][r, pl.ds(c * 16, 16)] = zero16
            return carry

        lax.fori_loop(0, ch, zrow, 0)
        for j in range(nz):
            pltpu.sync_copy(bufs[0], acc.at[pl.ds(sid * rpt + j * ch, ch)])
        plsc.subcore_barrier()

        def eloop(t, carry):
            g = t * nbuf
            descs = []
            for k in range(nbuf):
                for j in range(ch // 16):
                    p = packed_v[g + k, pl.ds(j * 16, 16)]
                    srcb[k][pl.ds(j * 16, 16)] = p & 0xFFFF
                    dstb[k][pl.ds(j * 16, 16)] = lax.shift_right_logical(p, 16)
                descs.append(
                    pltpu.async_copy(h_hbm.at[srcb[k]], bufs[k], sems[k]))
            for k in range(nbuf):
                descs[k].wait()
                pltpu.sync_copy(bufs[k], acc.at[dstb[k]], add=True)
            return carry

        lax.fori_loop(0, cpt // nbuf, eloop, 0)
        plsc.subcore_barrier()

        row0 = sid * rpt
        pltpu.sync_copy(acc.at[pl.ds(row0, rpt)],
                        out_hbm.at[pl.ds(cid * np_ + row0, rpt)])

    flat2 = seg(hcat, packed)
    return jnp.concatenate([flat2[:n], flat2[np_:np_ + n]], axis=0)


def kernel(x, edge_index, seed_idx, proj_W, proj_b, eps, lin1_W, lin1_b,
           lin2_W, lin2_b, bn_gamma, bn_beta, W1, W2, w_attn, cls_W, cls_b):
    n = x.shape[0]
    num_layers = eps.shape[0]
    ei = edge_index.astype(jnp.int32)
    src = ei[0]
    dst = ei[1]
    ch = 80
    e = src.shape[0]
    packed = (jnp.concatenate([src, src + n])
              | (jnp.concatenate([dst, dst]) << 16)).reshape(2 * e // ch, ch)
    seed = jnp.asarray(seed_idx, jnp.int32).reshape(1)

    hcat = _proj(x, proj_W, proj_b)
    for i in range(num_layers):
        flat = _segment_sum_sc(hcat, packed, n)
        hcat = _layer(hcat, flat, eps[i], lin1_W[i], lin1_b[i], lin2_W[i],
                      lin2_b[i], bn_gamma[i], bn_beta[i])

    alpha2, logit2 = _pool(seed, hcat, W1, W2, w_attn, cls_W, cls_b)
    return (logit2.reshape(1), alpha2[:, 0])
